# baseline (speedup 1.0000x reference)
import jax
import jax.numpy as jnp
import numpy as np
from jax import lax
from jax.experimental import pallas as pl
from jax.experimental.pallas import tpu as pltpu
from jax.experimental.pallas import tpu_sc as plsc

N_NODES = 100000
DIM = 16
N_TICKS = 51
ROW = DIM * N_TICKS
BATCH = 16384
NC = 2
NS = 16
NW = NC * NS
BPW = BATCH // NW
CH = 32
NCHUNK = BPW // CH
STEP = np.float32(1.0 / (N_TICKS - 1))


def _body(z2, bias16, src2, dst2, t2, out,
          src_v, dst_v, t_v, srows, drows, bias_v, out_v, sem_s, sem_d):
    wid = lax.axis_index("s") * NC + lax.axis_index("c")
    base = pl.multiple_of(wid * BPW, BPW)
    pltpu.sync_copy(bias16, bias_v)
    pltpu.sync_copy(src2.at[wid], src_v)
    pltpu.sync_copy(dst2.at[wid], dst_v)
    pltpu.sync_copy(t2.at[wid], t_v)

    def chunk(c, carry):
        cp_s = pltpu.async_copy(z2.at[src_v.at[c]], srows, sem_s)
        cp_d = pltpu.async_copy(z2.at[dst_v.at[c]], drows, sem_d)
        cp_s.wait()
        cp_d.wait()
        bias_vec = bias_v[...]
        for g in range(CH // 16):
            tv = t_v[pl.ds(c * CH + g * 16, 16)]
            ti = jnp.minimum((tv / STEP).astype(jnp.int32), N_TICKS - 2)
            dt = lax.rem(tv, STEP) / STEP
            omdt = 1.0 - dt
            rowi = lax.iota(jnp.int32, 16) + g * 16
            acc = jnp.zeros((16,), jnp.float32)
            for d in range(DIM):
                col = ti + d * N_TICKS
                scur = plsc.load_gather(srows, [rowi, col])
                snxt = plsc.load_gather(srows, [rowi, col + 1])
                dcur = plsc.load_gather(drows, [rowi, col])
                dnxt = plsc.load_gather(drows, [rowi, col + 1])
                diff = (omdt * scur + dt * snxt) - (omdt * dcur + dt * dnxt)
                acc = acc + diff * diff
            out_v[pl.ds(c * CH + g * 16, 16)] = bias_vec - acc
        return carry

    lax.fori_loop(0, NCHUNK, chunk, 0)
    pltpu.sync_copy(out_v, out.at[pl.ds(base, BPW)])


def kernel(z, bias, src, dst, t):
    z2 = z.reshape(N_NODES, ROW)
    bias16 = jnp.broadcast_to(jnp.asarray(bias, jnp.float32), (16,))
    src2 = src.astype(jnp.int32).reshape(NW, NCHUNK, CH)
    dst2 = dst.astype(jnp.int32).reshape(NW, NCHUNK, CH)
    t2 = t.reshape(NW, BPW)
    fn = pl.kernel(
        _body,
        out_type=jax.ShapeDtypeStruct((BATCH,), jnp.float32),
        mesh=plsc.VectorSubcoreMesh(core_axis_name="c", subcore_axis_name="s"),
        compiler_params=pltpu.CompilerParams(use_tc_tiling_on_sc=False,
                                             needs_layout_passes=False),
        scratch_types=[
            pltpu.VMEM((NCHUNK, CH), jnp.int32),
            pltpu.VMEM((NCHUNK, CH), jnp.int32),
            pltpu.VMEM((BPW,), jnp.float32),
            pltpu.VMEM((CH, ROW), jnp.float32),
            pltpu.VMEM((CH, ROW), jnp.float32),
            pltpu.VMEM((16,), jnp.float32),
            pltpu.VMEM((BPW,), jnp.float32),
            pltpu.SemaphoreType.DMA,
            pltpu.SemaphoreType.DMA,
        ],
    )
    return fn(z2, bias16, src2, dst2, t2)


# speedup vs baseline: 4.5130x; 4.5130x over previous
import jax
import jax.numpy as jnp
import numpy as np
from jax import lax
from jax.experimental import pallas as pl
from jax.experimental.pallas import tpu as pltpu
from jax.experimental.pallas import tpu_sc as plsc

N_NODES = 100000
DIM = 16
N_TICKS = 51
BATCH = 16384
NC = 2
NS = 16
NW = NC * NS
BPW = BATCH // NW
CH = 64
NCH = BPW // CH
NGRP = CH // 16
NROW = CH * 4 * DIM // 128
TSTRIDE = DIM * N_NODES
STEP = np.float32(1.0 / (N_TICKS - 1))


def _body(flat, bias16, src2, dst2, t2, out,
          src_v, dst_v, t_v, ti_v, dt_v,
          idx_a, idx_b, val_a, val_b, out_v, bias_v, sem_a, sem_b):
    wid = lax.axis_index("s") * NC + lax.axis_index("c")
    base = pl.multiple_of(wid * BPW, BPW)
    pltpu.sync_copy(bias16, bias_v)
    pltpu.sync_copy(src2.at[wid], src_v)
    pltpu.sync_copy(dst2.at[wid], dst_v)
    pltpu.sync_copy(t2.at[wid], t_v)

    def precomp(i, carry):
        tv = t_v[pl.ds(i * 16, 16)]
        ti_v[pl.ds(i * 16, 16)] = jnp.minimum((tv / STEP).astype(jnp.int32),
                                              N_TICKS - 2)
        dt_v[pl.ds(i * 16, 16)] = lax.rem(tv, STEP) / STEP
        return carry

    lax.fori_loop(0, BPW // 16, precomp, 0)

    def slot_store(idxb, s, v):
        idxb[s // 8, pl.ds((s % 8) * 16, 16)] = v

    def slot_load(vals, s):
        return vals[s // 8, pl.ds((s % 8) * 16, 16)]

    def build_issue(c, idxb, vals, sem):
        for grp in range(NGRP):
            g0 = c * CH + grp * 16
            tiv = ti_v[pl.ds(g0, 16)]
            sg = src_v[pl.ds(g0, 16)]
            dg = dst_v[pl.ds(g0, 16)]
            tb0 = tiv * TSTRIDE
            tb1 = tb0 + TSTRIDE
            for d in range(DIM):
                bs = sg + d * N_NODES
                bd = dg + d * N_NODES
                s = grp * 4 * DIM + d * 4
                slot_store(idxb, s + 0, tb0 + bs)
                slot_store(idxb, s + 1, tb1 + bs)
                slot_store(idxb, s + 2, tb0 + bd)
                slot_store(idxb, s + 3, tb1 + bd)
        for j in range(NROW):
            pltpu.async_copy(flat.at[idxb.at[j]], vals.at[j], sem)

    def drain_compute(c, idxb, vals, sem):
        for j in range(NROW):
            pltpu.make_async_copy(flat.at[idxb.at[j]], vals.at[j], sem).wait()
        bias_vec = bias_v[...]
        for grp in range(NGRP):
            g0 = c * CH + grp * 16
            dtg = dt_v[pl.ds(g0, 16)]
            acc = jnp.zeros((16,), jnp.float32)
            for d in range(DIM):
                s = grp * 4 * DIM + d * 4
                scur = slot_load(vals, s + 0)
                snxt = slot_load(vals, s + 1)
                dcur = slot_load(vals, s + 2)
                dnxt = slot_load(vals, s + 3)
                dc = scur - dcur
                dn = snxt - dnxt
                diff = dc + dtg * (dn - dc)
                acc = acc + diff * diff
            out_v[pl.ds(g0, 16)] = bias_vec - acc

    build_issue(0, idx_a, val_a, sem_a)

    def step2(i, carry):
        c0 = i * 2
        build_issue(c0 + 1, idx_b, val_b, sem_b)
        drain_compute(c0, idx_a, val_a, sem_a)
        build_issue(c0 + 2, idx_a, val_a, sem_a)
        drain_compute(c0 + 1, idx_b, val_b, sem_b)
        return carry

    lax.fori_loop(0, NCH // 2 - 1, step2, 0)
    build_issue(NCH - 1, idx_b, val_b, sem_b)
    drain_compute(NCH - 2, idx_a, val_a, sem_a)
    drain_compute(NCH - 1, idx_b, val_b, sem_b)
    pltpu.sync_copy(out_v, out.at[pl.ds(base, BPW)])


def kernel(z, bias, src, dst, t):
    flat = jnp.transpose(z, (2, 1, 0)).reshape(-1)
    bias16 = jnp.broadcast_to(jnp.asarray(bias, jnp.float32), (16,))
    src2 = src.astype(jnp.int32).reshape(NW, BPW)
    dst2 = dst.astype(jnp.int32).reshape(NW, BPW)
    t2 = t.reshape(NW, BPW)
    fn = pl.kernel(
        _body,
        out_type=jax.ShapeDtypeStruct((BATCH,), jnp.float32),
        mesh=plsc.VectorSubcoreMesh(core_axis_name="c", subcore_axis_name="s"),
        compiler_params=pltpu.CompilerParams(use_tc_tiling_on_sc=False,
                                             needs_layout_passes=False),
        scratch_types=[
            pltpu.VMEM((BPW,), jnp.int32),
            pltpu.VMEM((BPW,), jnp.int32),
            pltpu.VMEM((BPW,), jnp.float32),
            pltpu.VMEM((BPW,), jnp.int32),
            pltpu.VMEM((BPW,), jnp.float32),
            pltpu.VMEM((NROW, 128), jnp.int32),
            pltpu.VMEM((NROW, 128), jnp.int32),
            pltpu.VMEM((NROW, 128), jnp.float32),
            pltpu.VMEM((NROW, 128), jnp.float32),
            pltpu.VMEM((BPW,), jnp.float32),
            pltpu.VMEM((16,), jnp.float32),
            pltpu.SemaphoreType.DMA,
            pltpu.SemaphoreType.DMA,
        ],
    )
    return fn(flat, bias16, src2, dst2, t2)
